# baseline (speedup 1.0000x reference)
import functools

import jax
import jax.numpy as jnp
from jax import lax
from jax.experimental import pallas as pl
from jax.experimental.pallas import tpu as pltpu
from jax.experimental.pallas import tpu_sc as plsc

N = 10000
E = 320000
D = 128
DH = D // 2

NC = 2
NS = 16
NW = NC * NS
PER_W = E // NW
CH = 125
NCHUNK = PER_W // CH
PER_S = E // NS
NCHUNK2 = PER_S // CH
NP8 = 10240
RPT = NP8 // NS
ZR = 128
HL = 16

_mesh = plsc.VectorSubcoreMesh(core_axis_name="c", subcore_axis_name="s")


@functools.partial(
    pl.kernel,
    out_type=jax.ShapeDtypeStruct((NC, NP8, HL), jnp.float32),
    mesh=_mesh,
    scratch_types=[
        pltpu.VMEM((NCHUNK2 // 2, CH), jnp.int32),
        pltpu.VMEM((CH, HL), jnp.float32),
        pltpu.VMEM((RPT, HL), jnp.float32),
        pltpu.VMEM_SHARED((NP8, HL), jnp.float32),
        pltpu.SemaphoreType.DMA,
    ],
    compiler_params=pltpu.CompilerParams(use_tc_tiling_on_sc=False),
)
def _deg_sc(row_hbm, hist_hbm, idx_v, ones_v, zbuf, hist_sh, sem):
    c = lax.axis_index("c")
    s = lax.axis_index("s")

    one16 = jnp.full((HL,), 1.0, jnp.float32)
    zero16 = jnp.zeros((HL,), jnp.float32)

    @pl.loop(0, CH)
    def _(i):
        ones_v[i] = one16

    @pl.loop(0, RPT)
    def _(i):
        zbuf[i] = zero16

    pltpu.sync_copy(zbuf, hist_sh.at[pl.ds(s * RPT, RPT)])
    plsc.subcore_barrier()

    pltpu.sync_copy(row_hbm.at[s].at[pl.ds(c * (NCHUNK2 // 2), NCHUNK2 // 2)],
                    idx_v)

    @pl.loop(0, NCHUNK2 // 2)
    def _(ci):
        pltpu.async_copy(ones_v, hist_sh.at[idx_v.at[ci]], sem, add=True)

    @pl.loop(0, NCHUNK2 // 2)
    def _(ci):
        pltpu.make_async_copy(ones_v, hist_sh.at[idx_v.at[0]], sem).wait()

    plsc.subcore_barrier()
    pltpu.sync_copy(hist_sh.at[pl.ds(s * RPT, RPT)],
                    hist_hbm.at[c].at[pl.ds(s * RPT, RPT)])


@functools.partial(
    pl.kernel,
    out_type=jax.ShapeDtypeStruct((NC, NP8, DH), jnp.float32),
    mesh=_mesh,
    scratch_types=[
        pltpu.VMEM((NCHUNK2, CH), jnp.int32),
        pltpu.VMEM((NCHUNK2, CH), jnp.int32),
        pltpu.VMEM((CH, DH), jnp.float32),
        pltpu.VMEM((CH, DH), jnp.float32),
        pltpu.VMEM((CH, DH), jnp.float32),
        pltpu.VMEM((CH, DH), jnp.float32),
        pltpu.VMEM((ZR, DH), jnp.float32),
        pltpu.VMEM_SHARED((NP8, DH), jnp.float32),
        pltpu.SemaphoreType.DMA,
        pltpu.SemaphoreType.DMA,
        pltpu.SemaphoreType.DMA,
        pltpu.SemaphoreType.DMA,
        pltpu.SemaphoreType.DMA,
        pltpu.SemaphoreType.DMA,
        pltpu.SemaphoreType.DMA,
        pltpu.SemaphoreType.DMA,
    ],
    compiler_params=pltpu.CompilerParams(use_tc_tiling_on_sc=False),
)
def _spmm_sc(yl_hbm, yr_hbm, col_hbm, row_hbm, out_hbm, colv, rowv,
             b0, b1, b2, b3, zbuf, accum, g0, g1, g2, g3, s0, s1, s2, s3):
    bufs = (b0, b1, b2, b3)
    gsems = (g0, g1, g2, g3)
    ssems = (s0, s1, s2, s3)
    c = lax.axis_index("c")
    s = lax.axis_index("s")

    zero16 = jnp.zeros((16,), jnp.float32)

    @pl.loop(0, ZR)
    def _(i):
        @pl.loop(0, DH // 16)
        def _(j):
            zbuf[i, pl.ds(j * 16, 16)] = zero16

    @pl.loop(0, RPT // ZR)
    def _(k):
        pltpu.sync_copy(zbuf, accum.at[pl.ds(s * RPT + k * ZR, ZR)])

    plsc.subcore_barrier()

    pltpu.sync_copy(col_hbm.at[s], colv)
    pltpu.sync_copy(row_hbm.at[s], rowv)

    def run(y_hbm):
        for b in range(4):
            pltpu.async_copy(y_hbm.at[colv.at[b]], bufs[b], gsems[b])

        @pl.loop(0, NCHUNK2 // 4)
        def _(k):
            c_base = 4 * k
            for b in range(4):
                @pl.when(c_base + b < NCHUNK2)
                def _(b=b):
                    ci = c_base + b
                    pltpu.make_async_copy(y_hbm.at[colv.at[ci]], bufs[b],
                                          gsems[b]).wait()
                    pltpu.async_copy(bufs[b], accum.at[rowv.at[ci]], ssems[b],
                                     add=True)
            for b in range(4):
                @pl.when(c_base + b + 4 < NCHUNK2)
                def _(b=b):
                    ci = c_base + b + 4
                    pltpu.make_async_copy(bufs[b], accum.at[rowv.at[0]],
                                          ssems[b]).wait()
                    pltpu.async_copy(y_hbm.at[colv.at[ci]], bufs[b], gsems[b])

        for b in range(4):
            pltpu.make_async_copy(bufs[b], accum.at[rowv.at[0]], ssems[b]).wait()

    @pl.when(c == 0)
    def _():
        run(yl_hbm)

    @pl.when(c == 1)
    def _():
        run(yr_hbm)

    plsc.subcore_barrier()

    @pl.loop(0, RPT // ZR)
    def _(k):
        pltpu.sync_copy(accum.at[pl.ds(s * RPT + k * ZR, ZR)],
                        out_hbm.at[c].at[pl.ds(s * RPT + k * ZR, ZR)])


BLK = 2000


def _stage_a_body(hist_ref, x_ref, yl_ref, yr_ref, dinv_ref):
    deg = 1.0 + hist_ref[0, :, 0:1] + hist_ref[1, :, 0:1]
    dinv = lax.rsqrt(deg)
    y = x_ref[...] * dinv
    yl_ref[...] = y[:, :DH]
    yr_ref[...] = y[:, DH:]
    dinv_ref[...] = dinv


_stage_a = pl.pallas_call(
    _stage_a_body,
    grid=(N // BLK,),
    in_specs=[
        pl.BlockSpec((NC, BLK, HL), lambda i: (0, i, 0)),
        pl.BlockSpec((BLK, D), lambda i: (i, 0)),
    ],
    out_specs=[
        pl.BlockSpec((BLK, DH), lambda i: (i, 0)),
        pl.BlockSpec((BLK, DH), lambda i: (i, 0)),
        pl.BlockSpec((BLK, 1), lambda i: (i, 0)),
    ],
    out_shape=[
        jax.ShapeDtypeStruct((N, DH), jnp.float32),
        jax.ShapeDtypeStruct((N, DH), jnp.float32),
        jax.ShapeDtypeStruct((N, 1), jnp.float32),
    ],
)


def _stage_b_body(yl_ref, yr_ref, p_ref, dinv_ref, w1_ref, b1_ref,
                  y2l_ref, y2r_ref):
    zl = yl_ref[...] + p_ref[0]
    zr = yr_ref[...] + p_ref[1]
    u = jnp.concatenate([zl, zr], axis=1) * dinv_ref[...]
    h = lax.dot_general(u, w1_ref[...], (((1,), (1,)), ((), ())),
                        preferred_element_type=jnp.float32)
    h = jax.nn.gelu(h + b1_ref[...]) * dinv_ref[...]
    y2l_ref[...] = h[:, :DH]
    y2r_ref[...] = h[:, DH:]


_stage_b = pl.pallas_call(
    _stage_b_body,
    grid=(N // BLK,),
    in_specs=[
        pl.BlockSpec((BLK, DH), lambda i: (i, 0)),
        pl.BlockSpec((BLK, DH), lambda i: (i, 0)),
        pl.BlockSpec((NC, BLK, DH), lambda i: (0, i, 0)),
        pl.BlockSpec((BLK, 1), lambda i: (i, 0)),
        pl.BlockSpec((D, D), lambda i: (0, 0)),
        pl.BlockSpec((1, D), lambda i: (0, 0)),
    ],
    out_specs=[
        pl.BlockSpec((BLK, DH), lambda i: (i, 0)),
        pl.BlockSpec((BLK, DH), lambda i: (i, 0)),
    ],
    out_shape=[
        jax.ShapeDtypeStruct((N, DH), jnp.float32),
        jax.ShapeDtypeStruct((N, DH), jnp.float32),
    ],
)


def _stage_c_body(yl_ref, yr_ref, q_ref, dinv_ref, w2_ref, b2_ref,
                  out_ref):
    zl = yl_ref[...] + q_ref[0]
    zr = yr_ref[...] + q_ref[1]
    u = jnp.concatenate([zl, zr], axis=1) * dinv_ref[...]
    o = lax.dot_general(u, w2_ref[...], (((1,), (1,)), ((), ())),
                        preferred_element_type=jnp.float32)
    out_ref[...] = o + b2_ref[...]


_stage_c = pl.pallas_call(
    _stage_c_body,
    grid=(N // BLK,),
    in_specs=[
        pl.BlockSpec((BLK, DH), lambda i: (i, 0)),
        pl.BlockSpec((BLK, DH), lambda i: (i, 0)),
        pl.BlockSpec((NC, BLK, DH), lambda i: (0, i, 0)),
        pl.BlockSpec((BLK, 1), lambda i: (i, 0)),
        pl.BlockSpec((D, D), lambda i: (0, 0)),
        pl.BlockSpec((1, D), lambda i: (0, 0)),
    ],
    out_specs=pl.BlockSpec((BLK, D), lambda i: (i, 0)),
    out_shape=jax.ShapeDtypeStruct((N, D), jnp.float32),
)


def kernel(X, edge_index, W1, b1, W2, b2):
    row3s = edge_index[0].reshape(NS, NCHUNK2, CH)
    col3s = edge_index[1].reshape(NS, NCHUNK2, CH)
    b1r = b1.reshape(1, D)
    b2r = b2.reshape(1, D)

    hist = _deg_sc(row3s)
    y1l, y1r, dinv = _stage_a(hist, X)
    p = _spmm_sc(y1l, y1r, col3s, row3s)
    y2l, y2r = _stage_b(y1l, y1r, p, dinv, W1, b1r)
    q = _spmm_sc(y2l, y2r, col3s, row3s)
    out = _stage_c(y2l, y2r, q, dinv, W2, b2r)
    return out


# speedup vs baseline: 1.0035x; 1.0035x over previous
import functools

import jax
import jax.numpy as jnp
from jax import lax
from jax.experimental import pallas as pl
from jax.experimental.pallas import tpu as pltpu
from jax.experimental.pallas import tpu_sc as plsc

N = 10000
E = 320000
D = 128
DH = D // 2

NC = 2
NS = 16
NW = NC * NS
PER_W = E // NW
CH = 125
NCHUNK = PER_W // CH
PER_S = E // NS
NCHUNK2 = PER_S // CH
NP8 = 10240
RPT = NP8 // NS
ZR = 128
HL = 16

_mesh = plsc.VectorSubcoreMesh(core_axis_name="c", subcore_axis_name="s")


@functools.partial(
    pl.kernel,
    out_type=jax.ShapeDtypeStruct((NC, NP8, HL), jnp.float32),
    mesh=_mesh,
    scratch_types=[
        pltpu.VMEM((NCHUNK, CH), jnp.int32),
        pltpu.VMEM((CH, HL), jnp.float32),
        pltpu.VMEM((RPT, HL), jnp.float32),
        pltpu.VMEM_SHARED((NP8, HL), jnp.float32),
        pltpu.SemaphoreType.DMA,
    ],
    compiler_params=pltpu.CompilerParams(use_tc_tiling_on_sc=False),
)
def _deg_sc(row_hbm, hist_hbm, idx_v, ones_v, zbuf, hist_sh, sem):
    c = lax.axis_index("c")
    s = lax.axis_index("s")
    w = c * NS + s

    one16 = jnp.full((HL,), 1.0, jnp.float32)
    zero16 = jnp.zeros((HL,), jnp.float32)

    @pl.loop(0, CH)
    def _(i):
        ones_v[i] = one16

    @pl.loop(0, RPT)
    def _(i):
        zbuf[i] = zero16

    pltpu.sync_copy(zbuf, hist_sh.at[pl.ds(s * RPT, RPT)])
    plsc.subcore_barrier()

    pltpu.sync_copy(row_hbm.at[w], idx_v)

    @pl.loop(0, NCHUNK)
    def _(ci):
        pltpu.async_copy(ones_v, hist_sh.at[idx_v.at[ci]], sem, add=True)

    @pl.loop(0, NCHUNK)
    def _(ci):
        pltpu.make_async_copy(ones_v, hist_sh.at[idx_v.at[0]], sem).wait()

    plsc.subcore_barrier()
    pltpu.sync_copy(hist_sh.at[pl.ds(s * RPT, RPT)],
                    hist_hbm.at[c].at[pl.ds(s * RPT, RPT)])


@functools.partial(
    pl.kernel,
    out_type=jax.ShapeDtypeStruct((NC, NP8, DH), jnp.float32),
    mesh=_mesh,
    scratch_types=[
        pltpu.VMEM((NCHUNK2, CH), jnp.int32),
        pltpu.VMEM((NCHUNK2, CH), jnp.int32),
        pltpu.VMEM((CH, DH), jnp.float32),
        pltpu.VMEM((CH, DH), jnp.float32),
        pltpu.VMEM((CH, DH), jnp.float32),
        pltpu.VMEM((CH, DH), jnp.float32),
        pltpu.VMEM((ZR, DH), jnp.float32),
        pltpu.VMEM_SHARED((NP8, DH), jnp.float32),
        pltpu.SemaphoreType.DMA,
        pltpu.SemaphoreType.DMA,
        pltpu.SemaphoreType.DMA,
        pltpu.SemaphoreType.DMA,
        pltpu.SemaphoreType.DMA,
        pltpu.SemaphoreType.DMA,
        pltpu.SemaphoreType.DMA,
        pltpu.SemaphoreType.DMA,
    ],
    compiler_params=pltpu.CompilerParams(use_tc_tiling_on_sc=False),
)
def _spmm_sc(yl_hbm, yr_hbm, col_hbm, row_hbm, out_hbm, colv, rowv,
             b0, b1, b2, b3, zbuf, accum, g0, g1, g2, g3, s0, s1, s2, s3):
    bufs = (b0, b1, b2, b3)
    gsems = (g0, g1, g2, g3)
    ssems = (s0, s1, s2, s3)
    c = lax.axis_index("c")
    s = lax.axis_index("s")

    zero16 = jnp.zeros((16,), jnp.float32)

    @pl.loop(0, ZR)
    def _(i):
        @pl.loop(0, DH // 16)
        def _(j):
            zbuf[i, pl.ds(j * 16, 16)] = zero16

    @pl.loop(0, RPT // ZR)
    def _(k):
        pltpu.async_copy(zbuf, accum.at[pl.ds(s * RPT + k * ZR, ZR)], g0)

    @pl.loop(0, RPT // ZR)
    def _(k):
        pltpu.make_async_copy(zbuf, accum.at[pl.ds(0, ZR)], g0).wait()

    plsc.subcore_barrier()

    pltpu.sync_copy(col_hbm.at[s], colv)
    pltpu.sync_copy(row_hbm.at[s], rowv)

    def run(y_hbm):
        for b in range(4):
            pltpu.async_copy(y_hbm.at[colv.at[b]], bufs[b], gsems[b])

        @pl.loop(0, NCHUNK2 // 4)
        def _(k):
            c_base = 4 * k
            for b in range(4):
                @pl.when(c_base + b < NCHUNK2)
                def _(b=b):
                    ci = c_base + b
                    pltpu.make_async_copy(y_hbm.at[colv.at[ci]], bufs[b],
                                          gsems[b]).wait()
                    pltpu.async_copy(bufs[b], accum.at[rowv.at[ci]], ssems[b],
                                     add=True)
            for b in range(4):
                @pl.when(c_base + b + 4 < NCHUNK2)
                def _(b=b):
                    ci = c_base + b + 4
                    pltpu.make_async_copy(bufs[b], accum.at[rowv.at[0]],
                                          ssems[b]).wait()
                    pltpu.async_copy(y_hbm.at[colv.at[ci]], bufs[b], gsems[b])

        for b in range(4):
            pltpu.make_async_copy(bufs[b], accum.at[rowv.at[0]], ssems[b]).wait()

    @pl.when(c == 0)
    def _():
        run(yl_hbm)

    @pl.when(c == 1)
    def _():
        run(yr_hbm)

    plsc.subcore_barrier()

    @pl.loop(0, RPT // ZR)
    def _(k):
        pltpu.async_copy(accum.at[pl.ds(s * RPT + k * ZR, ZR)],
                         out_hbm.at[c].at[pl.ds(s * RPT + k * ZR, ZR)], g0)

    @pl.loop(0, RPT // ZR)
    def _(k):
        pltpu.make_async_copy(accum.at[pl.ds(0, ZR)],
                              out_hbm.at[c].at[pl.ds(0, ZR)], g0).wait()


BLK = 2000


def _stage_a_body(hist_ref, x_ref, yl_ref, yr_ref, dinv_ref):
    deg = 1.0 + hist_ref[0, :, 0:1] + hist_ref[1, :, 0:1]
    dinv = lax.rsqrt(deg)
    y = x_ref[...] * dinv
    yl_ref[...] = y[:, :DH]
    yr_ref[...] = y[:, DH:]
    dinv_ref[...] = dinv


_stage_a = pl.pallas_call(
    _stage_a_body,
    grid=(N // BLK,),
    in_specs=[
        pl.BlockSpec((NC, BLK, HL), lambda i: (0, i, 0)),
        pl.BlockSpec((BLK, D), lambda i: (i, 0)),
    ],
    out_specs=[
        pl.BlockSpec((BLK, DH), lambda i: (i, 0)),
        pl.BlockSpec((BLK, DH), lambda i: (i, 0)),
        pl.BlockSpec((BLK, 1), lambda i: (i, 0)),
    ],
    out_shape=[
        jax.ShapeDtypeStruct((N, DH), jnp.float32),
        jax.ShapeDtypeStruct((N, DH), jnp.float32),
        jax.ShapeDtypeStruct((N, 1), jnp.float32),
    ],
)


def _stage_b_body(yl_ref, yr_ref, p_ref, dinv_ref, w1_ref, b1_ref,
                  y2l_ref, y2r_ref):
    zl = yl_ref[...] + p_ref[0]
    zr = yr_ref[...] + p_ref[1]
    u = jnp.concatenate([zl, zr], axis=1) * dinv_ref[...]
    h = lax.dot_general(u, w1_ref[...], (((1,), (1,)), ((), ())),
                        preferred_element_type=jnp.float32)
    h = jax.nn.gelu(h + b1_ref[...]) * dinv_ref[...]
    y2l_ref[...] = h[:, :DH]
    y2r_ref[...] = h[:, DH:]


_stage_b = pl.pallas_call(
    _stage_b_body,
    grid=(N // BLK,),
    in_specs=[
        pl.BlockSpec((BLK, DH), lambda i: (i, 0)),
        pl.BlockSpec((BLK, DH), lambda i: (i, 0)),
        pl.BlockSpec((NC, BLK, DH), lambda i: (0, i, 0)),
        pl.BlockSpec((BLK, 1), lambda i: (i, 0)),
        pl.BlockSpec((D, D), lambda i: (0, 0)),
        pl.BlockSpec((1, D), lambda i: (0, 0)),
    ],
    out_specs=[
        pl.BlockSpec((BLK, DH), lambda i: (i, 0)),
        pl.BlockSpec((BLK, DH), lambda i: (i, 0)),
    ],
    out_shape=[
        jax.ShapeDtypeStruct((N, DH), jnp.float32),
        jax.ShapeDtypeStruct((N, DH), jnp.float32),
    ],
)


def _stage_c_body(yl_ref, yr_ref, q_ref, dinv_ref, w2_ref, b2_ref,
                  out_ref):
    zl = yl_ref[...] + q_ref[0]
    zr = yr_ref[...] + q_ref[1]
    u = jnp.concatenate([zl, zr], axis=1) * dinv_ref[...]
    o = lax.dot_general(u, w2_ref[...], (((1,), (1,)), ((), ())),
                        preferred_element_type=jnp.float32)
    out_ref[...] = o + b2_ref[...]


_stage_c = pl.pallas_call(
    _stage_c_body,
    grid=(N // BLK,),
    in_specs=[
        pl.BlockSpec((BLK, DH), lambda i: (i, 0)),
        pl.BlockSpec((BLK, DH), lambda i: (i, 0)),
        pl.BlockSpec((NC, BLK, DH), lambda i: (0, i, 0)),
        pl.BlockSpec((BLK, 1), lambda i: (i, 0)),
        pl.BlockSpec((D, D), lambda i: (0, 0)),
        pl.BlockSpec((1, D), lambda i: (0, 0)),
    ],
    out_specs=pl.BlockSpec((BLK, D), lambda i: (i, 0)),
    out_shape=jax.ShapeDtypeStruct((N, D), jnp.float32),
)


def kernel(X, edge_index, W1, b1, W2, b2):
    row3 = edge_index[0].reshape(NW, NCHUNK, CH)
    row3s = edge_index[0].reshape(NS, NCHUNK2, CH)
    col3s = edge_index[1].reshape(NS, NCHUNK2, CH)
    b1r = b1.reshape(1, D)
    b2r = b2.reshape(1, D)

    hist = _deg_sc(row3)
    y1l, y1r, dinv = _stage_a(hist, X)
    p = _spmm_sc(y1l, y1r, col3s, row3s)
    y2l, y2r = _stage_b(y1l, y1r, p, dinv, W1, b1r)
    q = _spmm_sc(y2l, y2r, col3s, row3s)
    out = _stage_c(y2l, y2r, q, dinv, W2, b2r)
    return out
